# Initial kernel scaffold; baseline (speedup 1.0000x reference)
import jax
import jax.numpy as jnp
from jax.experimental import pallas as pl


def kernel(src_tokens, table_boc, table_f, table_l, W1, b1, W2, b2):
    raise NotImplementedError("write your pallas kernel here")


# speedup vs baseline: 5.5429x; 5.5429x over previous
import functools

import jax
import jax.numpy as jnp
from jax import lax
from jax.experimental import pallas as pl
from jax.experimental.pallas import tpu as pltpu
from jax.experimental.pallas import tpu_sc as plsc

B, T, W = 1024, 200, 10
N = B * T
D = 128
NC, NS = 2, 16
NW = NC * NS
PER_TILE = N // NW
C = 64
CHUNKS = PER_TILE // C


def _sc_gather(table_f, table_l, table_boc, idx_f, idx_l, idx_boc):
    mesh = plsc.VectorSubcoreMesh(core_axis_name="c", subcore_axis_name="s")

    @functools.partial(
        pl.kernel,
        out_type=[jax.ShapeDtypeStruct((N, D), jnp.float32)] * 3,
        mesh=mesh,
        scratch_types=[
            pltpu.VMEM((C,), jnp.int32),
            pltpu.VMEM((C,), jnp.int32),
            pltpu.VMEM((8, C), jnp.int32),
            pltpu.VMEM((C, D), jnp.float32),
            pltpu.VMEM((C, D), jnp.float32),
            [pltpu.VMEM((C, D), jnp.float32) for _ in range(8)],
            pltpu.VMEM((C, D), jnp.float32),
            pltpu.SemaphoreType.DMA,
        ],
    )
    def sc_kernel(tf_h, tl_h, tb_h, idxf_h, idxl_h, idxb_h,
                  ef_h, eb_h, el_h,
                  idxf_v, idxl_v, idxb_v, buf_f, buf_l, bocs, ebuf, sem):
        wid = lax.axis_index("s") * NC + lax.axis_index("c")

        def chunk_body(k, carry):
            base = wid * PER_TILE + k * C
            pltpu.sync_copy(idxf_h.at[pl.ds(base, C)], idxf_v)
            pltpu.sync_copy(idxl_h.at[pl.ds(base, C)], idxl_v)
            for j in range(8):
                pltpu.sync_copy(idxb_h.at[j, pl.ds(base, C)], idxb_v.at[j])
            cps = [pltpu.async_copy(tf_h.at[idxf_v], buf_f, sem),
                   pltpu.async_copy(tl_h.at[idxl_v], buf_l, sem)]
            for j in range(8):
                cps.append(pltpu.async_copy(tb_h.at[idxb_v.at[j]], bocs[j], sem))
            for cp in cps:
                cp.wait()

            def tok_body(t, tc):
                for c in range(D // 16):
                    s = pl.ds(c * 16, 16)
                    v = bocs[0][t, s]
                    for j in range(1, 8):
                        v = v + bocs[j][t, s]
                    ebuf[t, s] = v
                return tc

            lax.fori_loop(0, C, tok_body, 0)
            pltpu.sync_copy(buf_f, ef_h.at[pl.ds(base, C)])
            pltpu.sync_copy(ebuf, eb_h.at[pl.ds(base, C)])
            pltpu.sync_copy(buf_l, el_h.at[pl.ds(base, C)])
            return carry

        lax.fori_loop(0, CHUNKS, chunk_body, 0)

    return sc_kernel(table_f, table_l, table_boc, idx_f, idx_l, idx_boc)


R = 1024


def _mlp_body(ef, eb, el, w1, b1, w2, b2, out):
    x = jnp.concatenate([ef[...], eb[...], el[...]], axis=1)
    h = jnp.maximum(jnp.dot(x, w1[...], preferred_element_type=jnp.float32)
                    + b1[...], 0.0)
    y = jnp.maximum(jnp.dot(h, w2[...], preferred_element_type=jnp.float32)
                    + b2[...], 0.0)
    out[...] = y


def _tc_mlp(ef, eb, el, w1, b1, w2, b2):
    grid = (N // R,)
    row_spec = pl.BlockSpec((R, D), lambda i: (i, 0))
    full = lambda shape: pl.BlockSpec(shape, lambda i: (0, 0))
    return pl.pallas_call(
        _mlp_body,
        grid=grid,
        in_specs=[row_spec, row_spec, row_spec,
                  full((3 * D, 3 * D)), full((1, 3 * D)),
                  full((3 * D, D)), full((1, D))],
        out_specs=row_spec,
        out_shape=jax.ShapeDtypeStruct((N, D), jnp.float32),
    )(ef, eb, el, w1, b1, w2, b2)


def kernel(src_tokens, table_boc, table_f, table_l, W1, b1, W2, b2):
    flat = src_tokens.reshape(N, W).astype(jnp.int32)
    idx_f = flat[:, 0]
    idx_l = flat[:, 1]
    idx_boc = flat[:, 2:].T
    ef, eb, el = _sc_gather(table_f, table_l, table_boc, idx_f, idx_l, idx_boc)
    w1 = jnp.concatenate([W1[:D], W1[D:2 * D] * (1.0 / 8.0), W1[2 * D:]], axis=0)
    out = _tc_mlp(ef, eb, el, w1, b1.reshape(1, -1), W2, b2.reshape(1, -1))
    return out.reshape(B, T, D)


# speedup vs baseline: 10.0217x; 1.8080x over previous
import functools

import jax
import jax.numpy as jnp
from jax import lax
from jax.experimental import pallas as pl
from jax.experimental.pallas import tpu as pltpu
from jax.experimental.pallas import tpu_sc as plsc

B, T, W = 1024, 200, 10
N = B * T
D = 128
NC, NS = 2, 16
NW = NC * NS
PER_TILE = N // NW
C = 32
CHUNKS = PER_TILE // C
IDXB = W * C


def _sc_gather(table_f, table_l, table_boc, idx_all):
    mesh = plsc.VectorSubcoreMesh(core_axis_name="c", subcore_axis_name="s")

    scratch = [
        [pltpu.VMEM((IDXB,), jnp.int32) for _ in range(2)],
        [pltpu.VMEM((C, D), jnp.float32) for _ in range(2)],
        [pltpu.VMEM((C, D), jnp.float32) for _ in range(2)],
        [[pltpu.VMEM((C, D), jnp.float32) for _ in range(8)] for _ in range(2)],
        [pltpu.VMEM((C, D), jnp.float32) for _ in range(2)],
        [pltpu.SemaphoreType.DMA for _ in range(2)],
        [pltpu.SemaphoreType.DMA for _ in range(2)],
        [pltpu.SemaphoreType.DMA for _ in range(2)],
    ]

    @functools.partial(
        pl.kernel,
        out_type=[jax.ShapeDtypeStruct((N, D), jnp.float32)] * 3,
        mesh=mesh,
        scratch_types=scratch,
    )
    def sc_kernel(tf_h, tl_h, tb_h, idx_h, ef_h, eb_h, el_h,
                  idxv, buf_f, buf_l, bocs, ebuf, sem_i, sem_g, sem_w):
        wid = lax.axis_index("s") * NC + lax.axis_index("c")
        blk0 = wid * CHUNKS

        def idx_copy(s, k):
            return pltpu.make_async_copy(idx_h.at[blk0 + k], idxv[s], sem_i[s])

        def gather_descs(s):
            ds_ = [pltpu.make_async_copy(tf_h.at[idxv[s].at[pl.ds(0, C)]],
                                         buf_f[s], sem_g[s]),
                   pltpu.make_async_copy(tl_h.at[idxv[s].at[pl.ds(C, C)]],
                                         buf_l[s], sem_g[s])]
            for j in range(8):
                ds_.append(pltpu.make_async_copy(
                    tb_h.at[idxv[s].at[pl.ds((2 + j) * C, C)]],
                    bocs[s][j], sem_g[s]))
            return ds_

        def write_descs(s, k):
            base = wid * PER_TILE + k * C
            return [pltpu.make_async_copy(buf_f[s], ef_h.at[pl.ds(base, C)], sem_w[s]),
                    pltpu.make_async_copy(ebuf[s], eb_h.at[pl.ds(base, C)], sem_w[s]),
                    pltpu.make_async_copy(buf_l[s], el_h.at[pl.ds(base, C)], sem_w[s])]

        idx_copy(0, 0).start()
        idx_copy(0, 0).wait()
        for d_ in gather_descs(0):
            d_.start()
        idx_copy(1, 1).start()

        def outer(i, carry):
            for b_ in range(2):
                k = 2 * i + b_
                s, s1 = b_, 1 - b_

                @pl.when(k >= 1)
                def _():
                    for d_ in write_descs(s1, 0):
                        d_.wait()

                @pl.when(k < CHUNKS - 1)
                def _():
                    idx_copy(s1, 0).wait()
                    for d_ in gather_descs(s1):
                        d_.start()

                for d_ in gather_descs(s):
                    d_.wait()

                @pl.when(k < CHUNKS - 2)
                def _():
                    idx_copy(s, k + 2).start()

                def tok(t, tc):
                    for c in range(D // 16):
                        sl = pl.ds(c * 16, 16)
                        v = bocs[s][0][t, sl]
                        for j in range(1, 8):
                            v = v + bocs[s][j][t, sl]
                        ebuf[s][t, sl] = v
                    return tc

                lax.fori_loop(0, C, tok, 0)
                for d_ in write_descs(s, k):
                    d_.start()
            return carry

        lax.fori_loop(0, CHUNKS // 2, outer, 0)
        for d_ in write_descs((CHUNKS - 1) % 2, 0):
            d_.wait()

    return sc_kernel(table_f, table_l, table_boc, idx_all)


R = 1024


def _mlp_body(ef, eb, el, w1, b1, w2, b2, out):
    x = jnp.concatenate([ef[...], eb[...], el[...]], axis=1)
    h = jnp.maximum(jnp.dot(x, w1[...], preferred_element_type=jnp.float32)
                    + b1[...], 0.0)
    y = jnp.maximum(jnp.dot(h, w2[...], preferred_element_type=jnp.float32)
                    + b2[...], 0.0)
    out[...] = y


def _tc_mlp(ef, eb, el, w1, b1, w2, b2):
    grid = (N // R,)
    row_spec = pl.BlockSpec((R, D), lambda i: (i, 0))
    full = lambda shape: pl.BlockSpec(shape, lambda i: (0, 0))
    return pl.pallas_call(
        _mlp_body,
        grid=grid,
        in_specs=[row_spec, row_spec, row_spec,
                  full((3 * D, 3 * D)), full((1, 3 * D)),
                  full((3 * D, D)), full((1, D))],
        out_specs=row_spec,
        out_shape=jax.ShapeDtypeStruct((N, D), jnp.float32),
    )(ef, eb, el, w1, b1, w2, b2)


def kernel(src_tokens, table_boc, table_f, table_l, W1, b1, W2, b2):
    flat = src_tokens.reshape(N, W).astype(jnp.int32)
    idx_all = (flat.reshape(NW, CHUNKS, C, W)
               .transpose(0, 1, 3, 2)
               .reshape(NW * CHUNKS, W * C))
    ef, eb, el = _sc_gather(table_f, table_l, table_boc, idx_all)
    w1 = jnp.concatenate([W1[:D], W1[D:2 * D] * (1.0 / 8.0), W1[2 * D:]], axis=0)
    out = _tc_mlp(ef, eb, el, w1, b1.reshape(1, -1), W2, b2.reshape(1, -1))
    return out.reshape(B, T, D)
